# Initial kernel scaffold; baseline (speedup 1.0000x reference)
import jax
import jax.numpy as jnp
from jax.experimental import pallas as pl


def kernel(positions, neighbors):
    raise NotImplementedError("write your pallas kernel here")


# speedup vs baseline: 114.5395x; 114.5395x over previous
import jax
import jax.numpy as jnp
from jax import lax
from jax.experimental import pallas as pl
from jax.experimental.pallas import tpu as pltpu
from jax.experimental.pallas import tpu_sc as plsc

NC, NS, L = 2, 16, 16
NW = NC * NS
B, NAT, NBH = 8, 4096, 32
ATOMS_PER_W = (B * NAT) // NW
PAIRS_PER_W = ATOMS_PER_W * NBH
WPB = NAT // ATOMS_PER_W


def _dist_body(pos_hbm, nbr_hbm, out_hbm, pos_v, nbr_v, out_v):
    wid = lax.axis_index("s") * NC + lax.axis_index("c")
    b = wid // WPB
    atom_base = (wid % WPB) * ATOMS_PER_W
    pair_base = wid * PAIRS_PER_W

    pltpu.sync_copy(pos_hbm.at[b], pos_v)
    pltpu.sync_copy(nbr_hbm.at[pl.ds(pair_base, PAIRS_PER_W)], nbr_v)

    iota = lax.iota(jnp.int32, L)

    def body(i, _):
        off = i * L
        nbr3 = nbr_v[pl.ds(off, L)] * 3
        a3 = (atom_base + lax.shift_right_logical(off + iota, 5)) * 3
        gx = plsc.load_gather(pos_v, [nbr3])
        gy = plsc.load_gather(pos_v, [nbr3 + 1])
        gz = plsc.load_gather(pos_v, [nbr3 + 2])
        cx = plsc.load_gather(pos_v, [a3])
        cy = plsc.load_gather(pos_v, [a3 + 1])
        cz = plsc.load_gather(pos_v, [a3 + 2])
        dx = gx - cx
        dy = gy - cy
        dz = gz - cz
        s = dx * dx + dy * dy + dz * dz
        bits = plsc.bitcast(s, jnp.int32)
        y = plsc.bitcast(
            0x5F3759DF - lax.shift_right_logical(bits, 1), jnp.float32)
        half_s = 0.5 * s
        y = y * (1.5 - half_s * y * y)
        y = y * (1.5 - half_s * y * y)
        y = y * (1.5 - half_s * y * y)
        d = jnp.where(s > 0.0, s * y, 0.0)
        out_v[pl.ds(off, L)] = d
        return 0

    lax.fori_loop(0, PAIRS_PER_W // L, body, 0)
    pltpu.sync_copy(out_v, out_hbm.at[pl.ds(pair_base, PAIRS_PER_W)])


def kernel(positions, neighbors):
    pos = positions.reshape(B, NAT * 3)
    nbr = neighbors.astype(jnp.int32).reshape(B * NAT * NBH)
    mesh = plsc.VectorSubcoreMesh(
        core_axis_name="c", subcore_axis_name="s",
        num_cores=NC, num_subcores=NS,
    )
    out = pl.kernel(
        _dist_body,
        out_type=jax.ShapeDtypeStruct((B * NAT * NBH,), jnp.float32),
        mesh=mesh,
        scratch_types=[
            pltpu.VMEM((NAT * 3,), jnp.float32),
            pltpu.VMEM((PAIRS_PER_W,), jnp.int32),
            pltpu.VMEM((PAIRS_PER_W,), jnp.float32),
        ],
        compiler_params=pltpu.CompilerParams(needs_layout_passes=False),
    )(pos, nbr)
    return out.reshape(B, NAT, NBH)


# speedup vs baseline: 141.5465x; 1.2358x over previous
import jax
import jax.numpy as jnp
from jax import lax
from jax.experimental import pallas as pl
from jax.experimental.pallas import tpu as pltpu
from jax.experimental.pallas import tpu_sc as plsc

NC, NS, L = 2, 16, 16
NW = NC * NS
B, NAT, NBH = 8, 4096, 32
ATOMS_PER_W = (B * NAT) // NW
PAIRS_PER_W = ATOMS_PER_W * NBH
WPB = NAT // ATOMS_PER_W
GROUPS = ATOMS_PER_W // L


def _dist_body(pos_hbm, nbr_hbm, out_hbm, pos_v, nbr_v, out_v):
    wid = lax.axis_index("s") * NC + lax.axis_index("c")
    b = wid // WPB
    atom_base = (wid % WPB) * ATOMS_PER_W
    pair_base = wid * PAIRS_PER_W

    pltpu.sync_copy(pos_hbm.at[b], pos_v)
    pltpu.sync_copy(nbr_hbm.at[pl.ds(pair_base, PAIRS_PER_W)], nbr_v)

    iota = lax.iota(jnp.int32, L)

    def group(g, _):
        a3 = (atom_base + g * L + iota) * 3
        cx = plsc.load_gather(pos_v, [a3])
        cy = plsc.load_gather(pos_v, [a3 + 1])
        cz = plsc.load_gather(pos_v, [a3 + 2])
        base_idx = g * (L * NBH) + iota * NBH

        @plsc.parallel_loop(0, NBH, unroll=4)
        def slot(j):
            idx = base_idx + j
            nbr3 = plsc.load_gather(nbr_v, [idx]) * 3
            gx = plsc.load_gather(pos_v, [nbr3])
            gy = plsc.load_gather(pos_v, [nbr3 + 1])
            gz = plsc.load_gather(pos_v, [nbr3 + 2])
            dx = gx - cx
            dy = gy - cy
            dz = gz - cz
            s = dx * dx + dy * dy + dz * dz
            bits = plsc.bitcast(s, jnp.int32)
            y = plsc.bitcast(
                0x5F3759DF - lax.shift_right_logical(bits, 1), jnp.float32)
            half_s = 0.5 * s
            y = y * (1.5 - half_s * y * y)
            y = y * (1.5 - half_s * y * y)
            d = jnp.where(s > 0.0, s * y, 0.0)
            plsc.store_scatter(out_v, [idx], d)

        return 0

    lax.fori_loop(0, GROUPS, group, 0)
    pltpu.sync_copy(out_v, out_hbm.at[pl.ds(pair_base, PAIRS_PER_W)])


def kernel(positions, neighbors):
    pos = positions.reshape(B, NAT * 3)
    nbr = neighbors.astype(jnp.int32).reshape(B * NAT * NBH)
    mesh = plsc.VectorSubcoreMesh(
        core_axis_name="c", subcore_axis_name="s",
        num_cores=NC, num_subcores=NS,
    )
    out = pl.kernel(
        _dist_body,
        out_type=jax.ShapeDtypeStruct((B * NAT * NBH,), jnp.float32),
        mesh=mesh,
        scratch_types=[
            pltpu.VMEM((NAT * 3,), jnp.float32),
            pltpu.VMEM((PAIRS_PER_W,), jnp.int32),
            pltpu.VMEM((PAIRS_PER_W,), jnp.float32),
        ],
        compiler_params=pltpu.CompilerParams(needs_layout_passes=False),
    )(pos, nbr)
    return out.reshape(B, NAT, NBH)


# speedup vs baseline: 180.5606x; 1.2756x over previous
import jax
import jax.numpy as jnp
from jax import lax
from jax.experimental import pallas as pl
from jax.experimental.pallas import tpu as pltpu
from jax.experimental.pallas import tpu_sc as plsc

NC, NS, L = 2, 16, 16
NW = NC * NS
B, NAT, NBH = 8, 4096, 32
ATOMS_PER_W = (B * NAT) // NW
WPB = NAT // ATOMS_PER_W
APR = 128 // NBH
ROWS_PER_W = ATOMS_PER_W // APR


def _dist_body(pos_hbm, nbr_hbm, out_hbm, pos_v, nbr_v, out_v):
    wid = lax.axis_index("s") * NC + lax.axis_index("c")
    b = wid // WPB
    atom_base = (wid % WPB) * ATOMS_PER_W
    row_base = wid * ROWS_PER_W

    pltpu.sync_copy(pos_hbm.at[pl.ds(b * NAT * 3, NAT * 3)],
                    pos_v.at[pl.ds(0, NAT * 3)])
    pltpu.sync_copy(nbr_hbm.at[pl.ds(row_base, ROWS_PER_W)], nbr_v)

    @plsc.parallel_loop(0, ATOMS_PER_W, unroll=4)
    def atom(a):
        r = lax.shift_right_logical(a, 2)
        cb = (a & (APR - 1)) * NBH
        a3 = (atom_base + a) * 3
        cv = pos_v[pl.ds(a3, L)]
        cx = cv[0]
        cy = cv[1]
        cz = cv[2]
        for h in range(NBH // L):
            nbr3 = nbr_v[r, pl.ds(cb + h * L, L)] * 3
            gx = plsc.load_gather(pos_v, [nbr3])
            gy = plsc.load_gather(pos_v, [nbr3 + 1])
            gz = plsc.load_gather(pos_v, [nbr3 + 2])
            dx = gx - cx
            dy = gy - cy
            dz = gz - cz
            s = dx * dx + dy * dy + dz * dz
            bits = plsc.bitcast(s, jnp.int32)
            y = plsc.bitcast(
                0x5F3759DF - lax.shift_right_logical(bits, 1), jnp.float32)
            half_s = 0.5 * s
            y = y * (1.5 - half_s * y * y)
            y = y * (1.5 - half_s * y * y)
            d = jnp.where(s > 0.0, s * y, 0.0)
            out_v[r, pl.ds(cb + h * L, L)] = d

    pltpu.sync_copy(out_v, out_hbm.at[pl.ds(row_base, ROWS_PER_W)])


def kernel(positions, neighbors):
    pos = positions.reshape(B * NAT * 3)
    nbr = neighbors.astype(jnp.int32).reshape(B * NAT * NBH // 128, 128)
    mesh = plsc.VectorSubcoreMesh(
        core_axis_name="c", subcore_axis_name="s",
        num_cores=NC, num_subcores=NS,
    )
    out = pl.kernel(
        _dist_body,
        out_type=jax.ShapeDtypeStruct((B * NAT * NBH // 128, 128),
                                      jnp.float32),
        mesh=mesh,
        scratch_types=[
            pltpu.VMEM((NAT * 3 + L,), jnp.float32),
            pltpu.VMEM((ROWS_PER_W, 128), jnp.int32),
            pltpu.VMEM((ROWS_PER_W, 128), jnp.float32),
        ],
        compiler_params=pltpu.CompilerParams(
            needs_layout_passes=False, use_tc_tiling_on_sc=True),
    )(pos, nbr)
    return out.reshape(B, NAT, NBH)


# speedup vs baseline: 209.0862x; 1.1580x over previous
import jax
import jax.numpy as jnp
from jax import lax
from jax.experimental import pallas as pl
from jax.experimental.pallas import tpu as pltpu
from jax.experimental.pallas import tpu_sc as plsc

NC, NS, L = 2, 16, 16
NW = NC * NS
B, NAT, NBH = 8, 4096, 32
ATOMS_PER_W = (B * NAT) // NW
WPB = NAT // ATOMS_PER_W
CHUNK = 256
NCHUNK = ATOMS_PER_W // CHUNK


def _dist_body(pos_hbm, nbr_hbm, out_hbm, pos_v, nbr_v, out_v):
    wid = lax.axis_index("s") * NC + lax.axis_index("c")
    b = wid // WPB
    atom_base = (wid % WPB) * ATOMS_PER_W
    row_base = wid * ATOMS_PER_W

    pltpu.sync_copy(pos_hbm.at[pl.ds(b * NAT * 3, NAT * 3)],
                    pos_v.at[pl.ds(0, NAT * 3)])

    def chunk(c, _):
        r0 = row_base + c * CHUNK
        pltpu.sync_copy(nbr_hbm.at[pl.ds(r0, CHUNK)], nbr_v)

        @plsc.parallel_loop(0, CHUNK, unroll=4)
        def atom(a):
            a3 = (atom_base + c * CHUNK + a) * 3
            cv = pos_v[pl.ds(a3, L)]
            cx = cv[0]
            cy = cv[1]
            cz = cv[2]
            for h in range(NBH // L):
                nbr3 = nbr_v[a, pl.ds(h * L, L)] * 3
                gx = plsc.load_gather(pos_v, [nbr3])
                gy = plsc.load_gather(pos_v, [nbr3 + 1])
                gz = plsc.load_gather(pos_v, [nbr3 + 2])
                dx = gx - cx
                dy = gy - cy
                dz = gz - cz
                s = dx * dx + dy * dy + dz * dz
                bits = plsc.bitcast(s, jnp.int32)
                y = plsc.bitcast(
                    0x5F3759DF - lax.shift_right_logical(bits, 1),
                    jnp.float32)
                half_s = 0.5 * s
                y = y * (1.5 - half_s * y * y)
                y = y * (1.5 - half_s * y * y)
                d = jnp.where(s > 0.0, s * y, 0.0)
                out_v[a, pl.ds(h * L, L)] = d

        pltpu.sync_copy(out_v, out_hbm.at[pl.ds(r0, CHUNK)])
        return 0

    lax.fori_loop(0, NCHUNK, chunk, 0)


def kernel(positions, neighbors):
    pos = positions.reshape(B * NAT * 3)
    nbr = neighbors.astype(jnp.int32).reshape(B * NAT, NBH)
    mesh = plsc.VectorSubcoreMesh(
        core_axis_name="c", subcore_axis_name="s",
        num_cores=NC, num_subcores=NS,
    )
    out = pl.kernel(
        _dist_body,
        out_type=jax.ShapeDtypeStruct((B * NAT, NBH), jnp.float32),
        mesh=mesh,
        scratch_types=[
            pltpu.VMEM((NAT * 3 + L,), jnp.float32),
            pltpu.VMEM((CHUNK, NBH), jnp.int32),
            pltpu.VMEM((CHUNK, NBH), jnp.float32),
        ],
        compiler_params=pltpu.CompilerParams(
            needs_layout_passes=False, use_tc_tiling_on_sc=True),
    )(pos, nbr)
    return out.reshape(B, NAT, NBH)


# speedup vs baseline: 229.0235x; 1.0954x over previous
import jax
import jax.numpy as jnp
from jax import lax
from jax.experimental import pallas as pl
from jax.experimental.pallas import tpu as pltpu
from jax.experimental.pallas import tpu_sc as plsc

NC, NS, L = 2, 16, 16
NW = NC * NS
B, NAT, NBH = 8, 4096, 32
ATOMS_PER_W = (B * NAT) // NW
WPB = NAT // ATOMS_PER_W
CHUNK = 128
NCHUNK = ATOMS_PER_W // CHUNK


def _dist_body(pos_hbm, nbr_hbm, out_hbm, pos_v, nbr_v0, nbr_v1,
               out_v0, out_v1, pos_sem, in_sem0, in_sem1,
               out_sem0, out_sem1):
    nbufs = (nbr_v0, nbr_v1)
    obufs = (out_v0, out_v1)
    isems = (in_sem0, in_sem1)
    osems = (out_sem0, out_sem1)
    wid = lax.axis_index("s") * NC + lax.axis_index("c")
    b = wid // WPB
    atom_base = (wid % WPB) * ATOMS_PER_W
    row_base = wid * ATOMS_PER_W

    pos_h = pltpu.async_copy(pos_hbm.at[pl.ds(b * NAT * 3, NAT * 3)],
                             pos_v.at[pl.ds(0, NAT * 3)], pos_sem)

    def start_in(c):
        return pltpu.async_copy(
            nbr_hbm.at[pl.ds(row_base + c * CHUNK, CHUNK)],
            nbufs[c % 2], isems[c % 2])

    def start_out(c):
        return pltpu.async_copy(
            obufs[c % 2],
            out_hbm.at[pl.ds(row_base + c * CHUNK, CHUNK)],
            osems[c % 2])

    handles_in = {0: start_in(0)}
    pos_h.wait()
    handles_out = {}
    for c in range(NCHUNK):
        if c + 1 < NCHUNK:
            handles_in[c + 1] = start_in(c + 1)
        handles_in.pop(c).wait()
        if c - 2 in handles_out:
            handles_out.pop(c - 2).wait()
        nv = nbufs[c % 2]
        ov = obufs[c % 2]
        cbase = (atom_base + c * CHUNK) * 3

        @plsc.parallel_loop(0, CHUNK, unroll=4)
        def atom(a):
            cv = pos_v[pl.ds(cbase + a * 3, L)]
            cx = cv[0]
            cy = cv[1]
            cz = cv[2]
            for h in range(NBH // L):
                nbr3 = nv[a, pl.ds(h * L, L)] * 3
                gx = plsc.load_gather(pos_v, [nbr3])
                gy = plsc.load_gather(pos_v, [nbr3 + 1])
                gz = plsc.load_gather(pos_v, [nbr3 + 2])
                dx = gx - cx
                dy = gy - cy
                dz = gz - cz
                s = dx * dx + dy * dy + dz * dz
                bits = plsc.bitcast(s, jnp.int32)
                y = plsc.bitcast(
                    0x5F3759DF - lax.shift_right_logical(bits, 1),
                    jnp.float32)
                half_s = 0.5 * s
                y = y * (1.5 - half_s * y * y)
                y = y * (1.5 - half_s * y * y)
                d = jnp.where(s > 0.0, s * y, 0.0)
                ov[a, pl.ds(h * L, L)] = d

        handles_out[c] = start_out(c)
    for c in sorted(handles_out):
        handles_out.pop(c).wait()


def kernel(positions, neighbors):
    pos = positions.reshape(B * NAT * 3)
    nbr = neighbors.astype(jnp.int32).reshape(B * NAT, NBH)
    mesh = plsc.VectorSubcoreMesh(
        core_axis_name="c", subcore_axis_name="s",
        num_cores=NC, num_subcores=NS,
    )
    out = pl.kernel(
        _dist_body,
        out_type=jax.ShapeDtypeStruct((B * NAT, NBH), jnp.float32),
        mesh=mesh,
        scratch_types=[
            pltpu.VMEM((NAT * 3 + L,), jnp.float32),
            pltpu.VMEM((CHUNK, NBH), jnp.int32),
            pltpu.VMEM((CHUNK, NBH), jnp.int32),
            pltpu.VMEM((CHUNK, NBH), jnp.float32),
            pltpu.VMEM((CHUNK, NBH), jnp.float32),
            pltpu.SemaphoreType.DMA,
            pltpu.SemaphoreType.DMA,
            pltpu.SemaphoreType.DMA,
            pltpu.SemaphoreType.DMA,
            pltpu.SemaphoreType.DMA,
        ],
        compiler_params=pltpu.CompilerParams(
            needs_layout_passes=False, use_tc_tiling_on_sc=True),
    )(pos, nbr)
    return out.reshape(B, NAT, NBH)
